# baseline (speedup 1.0000x reference)
import functools
import math

import jax
import jax.numpy as jnp
from jax import lax
from jax.experimental import pallas as pl
from jax.experimental.pallas import tpu as pltpu
from jax.experimental.pallas import tpu_sc as plsc

_NUM_SC = 2
_NUM_TEC = 16
_LANES = 16


def _score_body(q_ref, s_ref):
    q = q_ref[...]
    s = jnp.sum(q * q, axis=1)
    s_ref[...] = lax.bitcast_convert_type(s, jnp.int32).reshape(s_ref.shape)


def _topk_body(u, bh, T, U, s_hbm, idx_hbm, s_vmem, idx_vmem):
    rows = T // 128
    kper = 128 // _LANES
    wid = lax.axis_index("s") * _NUM_SC + lax.axis_index("c")

    heads_per_tile = (bh + _NUM_SC * _NUM_TEC - 1) // (_NUM_SC * _NUM_TEC)
    for rep in range(heads_per_tile):
        head = wid + rep * (_NUM_SC * _NUM_TEC)

        @pl.when(head < bh)
        def _process():
            pltpu.sync_copy(s_hbm.at[pl.ds(head * rows, rows)], s_vmem)

            zero = jnp.zeros((_LANES,), jnp.int32)

            def count_gt(mid):
                def body(r, acc):
                    for k in range(kper):
                        bits = s_vmem[r, pl.ds(k * _LANES, _LANES)]
                        acc = acc + (bits > mid).astype(jnp.int32)
                    return acc

                return jnp.sum(lax.fori_loop(0, rows, body, zero))

            def count3_gt(m1, m2, m3):
                def body(r, accs):
                    a1, a2, a3 = accs
                    for k in range(kper):
                        bits = s_vmem[r, pl.ds(k * _LANES, _LANES)]
                        a1 = a1 + (bits > m1).astype(jnp.int32)
                        a2 = a2 + (bits > m2).astype(jnp.int32)
                        a3 = a3 + (bits > m3).astype(jnp.int32)
                    return a1, a2, a3

                a1, a2, a3 = lax.fori_loop(0, rows, body, (zero, zero, zero))
                return jnp.sum(a1), jnp.sum(a2), jnp.sum(a3)

            def q_body(_, lo_hi):
                lo, hi = lo_hi
                step = (hi - lo) >> 2
                m1 = lo + step
                m2 = lo + 2 * step
                m3 = lo + 3 * step
                c1, c2, c3 = count3_gt(m1, m2, m3)
                lo2 = jnp.where(c3 >= u, m3,
                                jnp.where(c2 >= u, m2,
                                          jnp.where(c1 >= u, m1, lo)))
                hi2 = jnp.where(c1 < u, m1,
                                jnp.where(c2 < u, m2,
                                          jnp.where(c3 < u, m3, hi)))
                return lo2, hi2

            def b_body(_, lo_hi):
                lo, hi = lo_hi
                mid = lo + ((hi - lo) >> 1)
                big = count_gt(mid) >= u
                return (jnp.where(big, mid, lo), jnp.where(big, hi, mid))

            lo0 = jnp.int32(-1)
            hi0 = jnp.int32(0x7F800000)
            lohi = lax.fori_loop(0, 15, q_body, (lo0, hi0))
            _, thr = lax.fori_loop(0, 4, b_body, lohi)

            for j in range(U // _LANES):
                idx_vmem[pl.ds(j * _LANES, _LANES)] = jnp.full(
                    (_LANES,), -1, jnp.int32)

            def emit(r, off):
                for k in range(kper):
                    bits = s_vmem[r, pl.ds(k * _LANES, _LANES)]
                    m = bits >= thr
                    ones = jnp.where(m, 1, 0).astype(jnp.int32)
                    pos = off + plsc.cumsum(ones) - 1
                    vals = r * 128 + k * _LANES + lax.iota(jnp.int32, _LANES)
                    plsc.store_scatter(idx_vmem, [pos], vals,
                                       mask=m & (pos < U))
                    off = off + plsc.all_reduce_population_count(m)
                return off

            lax.fori_loop(0, rows, emit, jnp.zeros((_LANES,), jnp.int32))
            pltpu.sync_copy(idx_vmem, idx_hbm.at[head])


def _attn_body(scale, U, T, hpb, q_ref, k_ref, v_ref, ib_ref, o_ref):
    i = pl.program_id(0)
    for j in range(hpb):
        qt = q_ref[j]
        kt = k_ref[j]
        vt = v_ref[j]
        ib = ib_ref[pl.ds(i * hpb + j, 1), :]

        iota_tu = lax.broadcasted_iota(jnp.int32, (T, U), 0)
        onehot_t = (iota_tu == ib).astype(jnp.float32)

        q_sel = jnp.dot(qt, onehot_t,
                        preferred_element_type=jnp.float32) * scale
        s = lax.dot_general(kt, q_sel, (((0,), (0,)), ((), ())),
                            preferred_element_type=jnp.float32)
        m = jnp.max(s, axis=0, keepdims=True)
        e = jnp.exp(s - m)
        ones_t = jnp.ones((1, T), jnp.float32)
        denom = jnp.dot(ones_t, e,
                        preferred_element_type=jnp.float32)
        o_sel = jnp.dot(vt, e,
                        preferred_element_type=jnp.float32) / denom

        vmean = jnp.mean(vt, axis=1, keepdims=True)
        o_ref[j] = vmean + lax.dot_general(
            o_sel - vmean, onehot_t, (((1,), (1,)), ((), ())),
            preferred_element_type=jnp.float32)


def kernel(Q, K, V):
    B, H, T, D = Q.shape
    Tk = K.shape[2]
    u = max(1, min(T, int(math.ceil(math.log(Tk + 1) * 16))))
    U = ((u + 127) // 128) * 128
    BH = B * H
    scale = 1.0 / math.sqrt(D)

    Q2 = jnp.swapaxes(Q, 2, 3).reshape(BH, D, T)
    K2 = jnp.swapaxes(K, 2, 3).reshape(BH, D, T)
    V2 = jnp.swapaxes(V, 2, 3).reshape(BH, D, T)

    s_bits = pl.pallas_call(
        _score_body,
        grid=(BH // 8,),
        in_specs=[pl.BlockSpec((8, D, T), lambda i: (i, 0, 0))],
        out_specs=pl.BlockSpec((8 * T // 128, 128), lambda i: (i, 0)),
        out_shape=jax.ShapeDtypeStruct((BH * T // 128, 128), jnp.int32),
    )(Q2)

    mesh = plsc.VectorSubcoreMesh(core_axis_name="c", subcore_axis_name="s",
                                  num_cores=_NUM_SC, num_subcores=_NUM_TEC)
    topk = functools.partial(
        pl.kernel,
        out_type=jax.ShapeDtypeStruct((BH, U), jnp.int32),
        mesh=mesh,
        compiler_params=pltpu.CompilerParams(needs_layout_passes=False,
                                             use_tc_tiling_on_sc=True),
        scratch_types=[
            pltpu.VMEM((T // 128, 128), jnp.int32),
            pltpu.VMEM((U,), jnp.int32),
        ],
    )(functools.partial(_topk_body, u, BH, T, U))
    idx = topk(s_bits)

    hpb = 8
    out2 = pl.pallas_call(
        functools.partial(_attn_body, scale, U, T, hpb),
        grid=(BH // hpb,),
        in_specs=[
            pl.BlockSpec((hpb, D, T), lambda i: (i, 0, 0)),
            pl.BlockSpec((hpb, D, T), lambda i: (i, 0, 0)),
            pl.BlockSpec((hpb, D, T), lambda i: (i, 0, 0)),
            pl.BlockSpec((BH, U), lambda i: (0, 0)),
        ],
        out_specs=pl.BlockSpec((hpb, D, T), lambda i: (i, 0, 0)),
        out_shape=jax.ShapeDtypeStruct((BH, D, T), jnp.float32),
    )(Q2, K2, V2, idx)

    return jnp.swapaxes(out2.reshape(B, H, D, T), 2, 3)


# speedup vs baseline: 1.0415x; 1.0415x over previous
import functools
import math

import jax
import jax.numpy as jnp
from jax import lax
from jax.experimental import pallas as pl
from jax.experimental.pallas import tpu as pltpu
from jax.experimental.pallas import tpu_sc as plsc

_NUM_SC = 2
_NUM_TEC = 16
_LANES = 16


def _score_body(q_ref, s_ref):
    q = q_ref[...]
    s = jnp.sum(q * q, axis=1)
    s_ref[...] = lax.bitcast_convert_type(s, jnp.int32).reshape(s_ref.shape)


def _topk_body(u, bh, T, U, s_hbm, idx_hbm, s_vmem, idx_vmem):
    rows = T // 128
    kper = 128 // _LANES
    wid = lax.axis_index("s") * _NUM_SC + lax.axis_index("c")

    heads_per_tile = (bh + _NUM_SC * _NUM_TEC - 1) // (_NUM_SC * _NUM_TEC)
    for rep in range(heads_per_tile):
        head = wid + rep * (_NUM_SC * _NUM_TEC)

        @pl.when(head < bh)
        def _process():
            pltpu.sync_copy(s_hbm.at[pl.ds(head * rows, rows)], s_vmem)

            zero = jnp.zeros((_LANES,), jnp.int32)

            def count_gt(mid):
                def body(r, acc):
                    for k in range(kper):
                        bits = s_vmem[r, pl.ds(k * _LANES, _LANES)]
                        acc = acc + (bits > mid).astype(jnp.int32)
                    return acc

                return jnp.sum(lax.fori_loop(0, rows, body, zero))

            def count3_gt(m1, m2, m3):
                def body(r, accs):
                    a1, a2, a3 = accs
                    for k in range(kper):
                        bits = s_vmem[r, pl.ds(k * _LANES, _LANES)]
                        a1 = a1 + (bits > m1).astype(jnp.int32)
                        a2 = a2 + (bits > m2).astype(jnp.int32)
                        a3 = a3 + (bits > m3).astype(jnp.int32)
                    return a1, a2, a3

                a1, a2, a3 = lax.fori_loop(0, rows, body, (zero, zero, zero))
                return jnp.sum(a1), jnp.sum(a2), jnp.sum(a3)

            def q_body(_, lo_hi):
                lo, hi = lo_hi
                step = (hi - lo) >> 2
                m1 = lo + step
                m2 = lo + 2 * step
                m3 = lo + 3 * step
                c1, c2, c3 = count3_gt(m1, m2, m3)
                lo2 = jnp.where(c3 >= u, m3,
                                jnp.where(c2 >= u, m2,
                                          jnp.where(c1 >= u, m1, lo)))
                hi2 = jnp.where(c1 < u, m1,
                                jnp.where(c2 < u, m2,
                                          jnp.where(c3 < u, m3, hi)))
                return lo2, hi2

            def b_body(_, lo_hi):
                lo, hi = lo_hi
                mid = lo + ((hi - lo) >> 1)
                big = count_gt(mid) >= u
                return (jnp.where(big, mid, lo), jnp.where(big, hi, mid))

            lo0 = jnp.int32(-1)
            hi0 = jnp.int32(0x7F800000)
            lohi = lax.fori_loop(0, 15, q_body, (lo0, hi0))
            _, thr = lax.fori_loop(0, 4, b_body, lohi)

            for j in range(U // _LANES):
                idx_vmem[pl.ds(j * _LANES, _LANES)] = jnp.full(
                    (_LANES,), -1, jnp.int32)

            def emit(r, off):
                for k in range(kper):
                    bits = s_vmem[r, pl.ds(k * _LANES, _LANES)]
                    m = bits >= thr
                    ones = jnp.where(m, 1, 0).astype(jnp.int32)
                    pos = off + plsc.cumsum(ones) - 1
                    vals = r * 128 + k * _LANES + lax.iota(jnp.int32, _LANES)
                    plsc.store_scatter(idx_vmem, [pos], vals,
                                       mask=m & (pos < U))
                    off = off + plsc.all_reduce_population_count(m)
                return off

            lax.fori_loop(0, rows, emit, jnp.zeros((_LANES,), jnp.int32))
            pltpu.sync_copy(idx_vmem, idx_hbm.at[head])


def _attn_body(scale, U, T, hpb, q_ref, k_ref, v_ref, ib_ref, o_ref):
    i = pl.program_id(0)
    for j in range(hpb):
        qt = q_ref[j]
        kt = k_ref[j]
        vt = v_ref[j]
        ib = ib_ref[pl.ds(i * hpb + j, 1), :]

        iota_tu = lax.broadcasted_iota(jnp.int32, (T, U), 0)
        onehot_t = (iota_tu == ib).astype(jnp.float32)

        q_sel = jnp.dot(qt, onehot_t,
                        preferred_element_type=jnp.float32) * scale
        s = lax.dot_general(kt, q_sel, (((0,), (0,)), ((), ())),
                            preferred_element_type=jnp.float32)
        m = jnp.max(s, axis=0, keepdims=True)
        e = jnp.exp(s - m)
        ones_t = jnp.ones((1, T), jnp.float32)
        denom = jnp.dot(ones_t, e,
                        preferred_element_type=jnp.float32)
        o_sel = jnp.dot(vt, e,
                        preferred_element_type=jnp.float32) / denom

        vmean = jnp.mean(vt, axis=1, keepdims=True)
        o_ref[j] = vmean + lax.dot_general(
            o_sel - vmean, onehot_t, (((1,), (1,)), ((), ())),
            preferred_element_type=jnp.float32)


def kernel(Q, K, V):
    B, H, T, D = Q.shape
    Tk = K.shape[2]
    u = max(1, min(T, int(math.ceil(math.log(Tk + 1) * 16))))
    U = ((u + 127) // 128) * 128
    BH = B * H
    scale = 1.0 / math.sqrt(D)

    Q2 = jnp.swapaxes(Q, 2, 3).reshape(BH, D, T)
    K2 = jnp.swapaxes(K, 2, 3).reshape(BH, D, T)
    V2 = jnp.swapaxes(V, 2, 3).reshape(BH, D, T)

    s_bits = pl.pallas_call(
        _score_body,
        grid=(BH // 8,),
        in_specs=[pl.BlockSpec((8, D, T), lambda i: (i, 0, 0))],
        out_specs=pl.BlockSpec((8 * T // 128, 128), lambda i: (i, 0)),
        out_shape=jax.ShapeDtypeStruct((BH * T // 128, 128), jnp.int32),
    )(Q2)

    mesh = plsc.VectorSubcoreMesh(core_axis_name="c", subcore_axis_name="s",
                                  num_cores=_NUM_SC, num_subcores=_NUM_TEC)
    topk = functools.partial(
        pl.kernel,
        out_type=jax.ShapeDtypeStruct((BH, U), jnp.int32),
        mesh=mesh,
        compiler_params=pltpu.CompilerParams(needs_layout_passes=False,
                                             use_tc_tiling_on_sc=True),
        scratch_types=[
            pltpu.VMEM((T // 128, 128), jnp.int32),
            pltpu.VMEM((U,), jnp.int32),
        ],
    )(functools.partial(_topk_body, u, BH, T, U))
    idx = topk(s_bits)

    hpb = 4
    out2 = pl.pallas_call(
        functools.partial(_attn_body, scale, U, T, hpb),
        grid=(BH // hpb,),
        in_specs=[
            pl.BlockSpec((hpb, D, T), lambda i: (i, 0, 0)),
            pl.BlockSpec((hpb, D, T), lambda i: (i, 0, 0)),
            pl.BlockSpec((hpb, D, T), lambda i: (i, 0, 0)),
            pl.BlockSpec((BH, U), lambda i: (0, 0)),
        ],
        out_specs=pl.BlockSpec((hpb, D, T), lambda i: (i, 0, 0)),
        out_shape=jax.ShapeDtypeStruct((BH, D, T), jnp.float32),
    )(Q2, K2, V2, idx)

    return jnp.swapaxes(out2.reshape(B, H, D, T), 2, 3)


# speedup vs baseline: 1.0727x; 1.0299x over previous
import functools
import math

import jax
import jax.numpy as jnp
from jax import lax
from jax.experimental import pallas as pl
from jax.experimental.pallas import tpu as pltpu
from jax.experimental.pallas import tpu_sc as plsc

_NUM_SC = 2
_NUM_TEC = 16
_LANES = 16


def _score_body(q_ref, s_ref):
    q = q_ref[...]
    s = jnp.sum(q * q, axis=1)
    s_ref[...] = lax.bitcast_convert_type(s, jnp.int32).reshape(s_ref.shape)


def _topk_body(u, bh, T, U, s_hbm, idx_hbm, s_vmem, idx_vmem):
    rows = T // 128
    kper = 128 // _LANES
    wid = lax.axis_index("s") * _NUM_SC + lax.axis_index("c")

    heads_per_tile = (bh + _NUM_SC * _NUM_TEC - 1) // (_NUM_SC * _NUM_TEC)
    for rep in range(heads_per_tile):
        head = wid + rep * (_NUM_SC * _NUM_TEC)

        @pl.when(head < bh)
        def _process():
            pltpu.sync_copy(s_hbm.at[pl.ds(head * rows, rows)], s_vmem)

            zero = jnp.zeros((_LANES,), jnp.int32)

            def count_gt(mid):
                def body(r, acc):
                    for k in range(kper):
                        bits = s_vmem[r, pl.ds(k * _LANES, _LANES)]
                        acc = acc + (bits > mid).astype(jnp.int32)
                    return acc

                return jnp.sum(lax.fori_loop(0, rows, body, zero))

            def count3_gt(m1, m2, m3):
                def body(r, accs):
                    a1, a2, a3 = accs
                    for k in range(kper):
                        bits = s_vmem[r, pl.ds(k * _LANES, _LANES)]
                        a1 = a1 + (bits > m1).astype(jnp.int32)
                        a2 = a2 + (bits > m2).astype(jnp.int32)
                        a3 = a3 + (bits > m3).astype(jnp.int32)
                    return a1, a2, a3

                a1, a2, a3 = lax.fori_loop(0, rows, body, (zero, zero, zero))
                return jnp.sum(a1), jnp.sum(a2), jnp.sum(a3)

            def q_body(_, lo_hi):
                lo, hi = lo_hi
                step = (hi - lo) >> 2
                m1 = lo + step
                m2 = lo + 2 * step
                m3 = lo + 3 * step
                c1, c2, c3 = count3_gt(m1, m2, m3)
                lo2 = jnp.where(c3 >= u, m3,
                                jnp.where(c2 >= u, m2,
                                          jnp.where(c1 >= u, m1, lo)))
                hi2 = jnp.where(c1 < u, m1,
                                jnp.where(c2 < u, m2,
                                          jnp.where(c3 < u, m3, hi)))
                return lo2, hi2

            def b_body(_, lo_hi):
                lo, hi = lo_hi
                mid = lo + ((hi - lo) >> 1)
                big = count_gt(mid) >= u
                return (jnp.where(big, mid, lo), jnp.where(big, hi, mid))

            lo0 = jnp.int32(-1)
            hi0 = jnp.int32(0x7F800000)
            lohi = lax.fori_loop(0, 15, q_body, (lo0, hi0))
            _, thr = lax.fori_loop(0, 4, b_body, lohi)

            for j in range(U // _LANES):
                idx_vmem[pl.ds(j * _LANES, _LANES)] = jnp.full(
                    (_LANES,), -1, jnp.int32)

            def emit(r, off):
                for k in range(kper):
                    bits = s_vmem[r, pl.ds(k * _LANES, _LANES)]
                    m = bits >= thr
                    ones = jnp.where(m, 1, 0).astype(jnp.int32)
                    pos = off + plsc.cumsum(ones) - 1
                    vals = r * 128 + k * _LANES + lax.iota(jnp.int32, _LANES)
                    plsc.store_scatter(idx_vmem, [pos], vals,
                                       mask=m & (pos < U))
                    off = off + plsc.all_reduce_population_count(m)
                return off

            lax.fori_loop(0, rows, emit, jnp.zeros((_LANES,), jnp.int32))
            pltpu.sync_copy(idx_vmem, idx_hbm.at[head])


def _attn_body(scale, U, T, hpb, q_ref, k_ref, v_ref, ib_ref, o_ref):
    i = pl.program_id(0)
    for j in range(hpb):
        qt = q_ref[j]
        kt = k_ref[j]
        vt = v_ref[j]
        ib = ib_ref[pl.ds(i * hpb + j, 1), :]

        iota_tu = lax.broadcasted_iota(jnp.int32, (T, U), 0)
        onehot_t = (iota_tu == ib).astype(jnp.float32)

        q_sel = jnp.dot(qt, onehot_t,
                        preferred_element_type=jnp.float32)
        s = lax.dot_general(kt, q_sel, (((0,), (0,)), ((), ())),
                            preferred_element_type=jnp.float32) * scale
        m = jnp.max(s, axis=0, keepdims=True)
        e = jnp.exp(s - m)
        ones_t = jnp.ones((1, T), jnp.float32)
        denom = jnp.dot(ones_t, e,
                        preferred_element_type=jnp.float32)
        o_sel = jnp.dot(vt, e,
                        preferred_element_type=jnp.float32) / denom

        vmean = jnp.mean(vt, axis=1, keepdims=True)
        o_ref[j] = vmean + lax.dot_general(
            o_sel - vmean, onehot_t, (((1,), (1,)), ((), ())),
            preferred_element_type=jnp.float32)


def kernel(Q, K, V):
    B, H, T, D = Q.shape
    Tk = K.shape[2]
    u = max(1, min(T, int(math.ceil(math.log(Tk + 1) * 16))))
    U = ((u + 127) // 128) * 128
    BH = B * H
    scale = 1.0 / math.sqrt(D)

    Q2 = jnp.swapaxes(Q, 2, 3).reshape(BH, D, T)
    K2 = jnp.swapaxes(K, 2, 3).reshape(BH, D, T)
    V2 = jnp.swapaxes(V, 2, 3).reshape(BH, D, T)

    s_bits = pl.pallas_call(
        _score_body,
        grid=(BH // 8,),
        in_specs=[pl.BlockSpec((8, D, T), lambda i: (i, 0, 0))],
        out_specs=pl.BlockSpec((8 * T // 128, 128), lambda i: (i, 0)),
        out_shape=jax.ShapeDtypeStruct((BH * T // 128, 128), jnp.int32),
    )(Q2)

    mesh = plsc.VectorSubcoreMesh(core_axis_name="c", subcore_axis_name="s",
                                  num_cores=_NUM_SC, num_subcores=_NUM_TEC)
    topk = functools.partial(
        pl.kernel,
        out_type=jax.ShapeDtypeStruct((BH, U), jnp.int32),
        mesh=mesh,
        compiler_params=pltpu.CompilerParams(needs_layout_passes=False,
                                             use_tc_tiling_on_sc=True),
        scratch_types=[
            pltpu.VMEM((T // 128, 128), jnp.int32),
            pltpu.VMEM((U,), jnp.int32),
        ],
    )(functools.partial(_topk_body, u, BH, T, U))
    idx = topk(s_bits)

    hpb = 4
    out2 = pl.pallas_call(
        functools.partial(_attn_body, scale, U, T, hpb),
        grid=(BH // hpb,),
        in_specs=[
            pl.BlockSpec((hpb, D, T), lambda i: (i, 0, 0)),
            pl.BlockSpec((hpb, D, T), lambda i: (i, 0, 0)),
            pl.BlockSpec((hpb, D, T), lambda i: (i, 0, 0)),
            pl.BlockSpec((BH, U), lambda i: (0, 0)),
        ],
        out_specs=pl.BlockSpec((hpb, D, T), lambda i: (i, 0, 0)),
        out_shape=jax.ShapeDtypeStruct((BH, D, T), jnp.float32),
    )(Q2, K2, V2, idx)

    return jnp.swapaxes(out2.reshape(B, H, D, T), 2, 3)


# speedup vs baseline: 1.1125x; 1.0371x over previous
import functools
import math

import jax
import jax.numpy as jnp
from jax import lax
from jax.experimental import pallas as pl
from jax.experimental.pallas import tpu as pltpu
from jax.experimental.pallas import tpu_sc as plsc

_NUM_SC = 2
_NUM_TEC = 16
_LANES = 16


def _score_body(q_ref, s_ref):
    q = q_ref[...]
    s = jnp.sum(q * q, axis=1)
    s_ref[...] = lax.bitcast_convert_type(s, jnp.int32).reshape(s_ref.shape)


def _topk_body(u, bh, T, U, s_hbm, idx_hbm, s_vmem, idx_vmem):
    rows = T // 128
    kper = 128 // _LANES
    wid = lax.axis_index("s") * _NUM_SC + lax.axis_index("c")

    heads_per_tile = (bh + _NUM_SC * _NUM_TEC - 1) // (_NUM_SC * _NUM_TEC)
    for rep in range(heads_per_tile):
        head = wid + rep * (_NUM_SC * _NUM_TEC)

        @pl.when(head < bh)
        def _process():
            pltpu.sync_copy(s_hbm.at[pl.ds(head * rows, rows)], s_vmem)

            zero = jnp.zeros((_LANES,), jnp.int32)

            def count_gt(mid):
                def body(r, acc):
                    for k in range(kper):
                        bits = s_vmem[r, pl.ds(k * _LANES, _LANES)]
                        acc = acc + (bits > mid).astype(jnp.int32)
                    return acc

                return jnp.sum(lax.fori_loop(0, rows, body, zero))

            def b_body(_, lo_hi):
                lo, hi = lo_hi
                mid = lo + ((hi - lo) >> 1)
                big = count_gt(mid) >= u
                return (jnp.where(big, mid, lo), jnp.where(big, hi, mid))

            lo0 = jnp.int32(-1)
            hi0 = jnp.int32(0x7F800000)
            _, thr = lax.fori_loop(0, 32, b_body, (lo0, hi0))

            for j in range(U // _LANES):
                idx_vmem[pl.ds(j * _LANES, _LANES)] = jnp.full(
                    (_LANES,), -1, jnp.int32)

            def emit(r, off):
                for k in range(kper):
                    bits = s_vmem[r, pl.ds(k * _LANES, _LANES)]
                    m = bits >= thr
                    ones = jnp.where(m, 1, 0).astype(jnp.int32)
                    pos = off + plsc.cumsum(ones) - 1
                    vals = r * 128 + k * _LANES + lax.iota(jnp.int32, _LANES)
                    plsc.store_scatter(idx_vmem, [pos], vals,
                                       mask=m & (pos < U))
                    off = off + plsc.all_reduce_population_count(m)
                return off

            lax.fori_loop(0, rows, emit, jnp.zeros((_LANES,), jnp.int32))
            pltpu.sync_copy(idx_vmem, idx_hbm.at[head])


def _attn_body(scale, U, T, hpb, q_ref, k_ref, v_ref, ib_ref, o_ref):
    i = pl.program_id(0)
    for j in range(hpb):
        qt = q_ref[j]
        kt = k_ref[j]
        vt = v_ref[j]
        ib = ib_ref[pl.ds(i * hpb + j, 1), :]

        iota_tu = lax.broadcasted_iota(jnp.int32, (T, U), 0)
        onehot_t = (iota_tu == ib).astype(jnp.float32)

        q_sel = jnp.dot(qt, onehot_t,
                        preferred_element_type=jnp.float32)
        s = lax.dot_general(kt, q_sel, (((0,), (0,)), ((), ())),
                            preferred_element_type=jnp.float32) * scale
        m = jnp.max(s, axis=0, keepdims=True)
        e = jnp.exp(s - m)
        ones_t = jnp.ones((1, T), jnp.float32)
        denom = jnp.dot(ones_t, e,
                        preferred_element_type=jnp.float32)
        o_sel = jnp.dot(vt, e,
                        preferred_element_type=jnp.float32) / denom

        vmean = jnp.mean(vt, axis=1, keepdims=True)
        o_ref[j] = vmean + lax.dot_general(
            o_sel - vmean, onehot_t, (((1,), (1,)), ((), ())),
            preferred_element_type=jnp.float32)


def kernel(Q, K, V):
    B, H, T, D = Q.shape
    Tk = K.shape[2]
    u = max(1, min(T, int(math.ceil(math.log(Tk + 1) * 16))))
    U = ((u + 127) // 128) * 128
    BH = B * H
    scale = 1.0 / math.sqrt(D)

    Q2 = jnp.swapaxes(Q, 2, 3).reshape(BH, D, T)
    K2 = jnp.swapaxes(K, 2, 3).reshape(BH, D, T)
    V2 = jnp.swapaxes(V, 2, 3).reshape(BH, D, T)

    s_bits = pl.pallas_call(
        _score_body,
        grid=(BH // 8,),
        in_specs=[pl.BlockSpec((8, D, T), lambda i: (i, 0, 0))],
        out_specs=pl.BlockSpec((8 * T // 128, 128), lambda i: (i, 0)),
        out_shape=jax.ShapeDtypeStruct((BH * T // 128, 128), jnp.int32),
    )(Q2)

    mesh = plsc.VectorSubcoreMesh(core_axis_name="c", subcore_axis_name="s",
                                  num_cores=_NUM_SC, num_subcores=_NUM_TEC)
    topk = functools.partial(
        pl.kernel,
        out_type=jax.ShapeDtypeStruct((BH, U), jnp.int32),
        mesh=mesh,
        compiler_params=pltpu.CompilerParams(needs_layout_passes=False,
                                             use_tc_tiling_on_sc=True),
        scratch_types=[
            pltpu.VMEM((T // 128, 128), jnp.int32),
            pltpu.VMEM((U,), jnp.int32),
        ],
    )(functools.partial(_topk_body, u, BH, T, U))
    idx = topk(s_bits)

    hpb = 4
    out2 = pl.pallas_call(
        functools.partial(_attn_body, scale, U, T, hpb),
        grid=(BH // hpb,),
        in_specs=[
            pl.BlockSpec((hpb, D, T), lambda i: (i, 0, 0)),
            pl.BlockSpec((hpb, D, T), lambda i: (i, 0, 0)),
            pl.BlockSpec((hpb, D, T), lambda i: (i, 0, 0)),
            pl.BlockSpec((BH, U), lambda i: (0, 0)),
        ],
        out_specs=pl.BlockSpec((hpb, D, T), lambda i: (i, 0, 0)),
        out_shape=jax.ShapeDtypeStruct((BH, D, T), jnp.float32),
    )(Q2, K2, V2, idx)

    return jnp.swapaxes(out2.reshape(B, H, D, T), 2, 3)
